# baseline (speedup 1.0000x reference)
import functools

import jax
import jax.numpy as jnp
from jax import lax
from jax.experimental import pallas as pl
from jax.experimental.pallas import tpu as pltpu
from jax.experimental.pallas import tpu_sc as plsc

NC, NS = 2, 16
NW = NC * NS
LANES = 16


def _make_lookup(batch, seq_len, latent_dim):
    ch = seq_len // NW
    nsl = latent_dim // LANES
    mesh = plsc.VectorSubcoreMesh(core_axis_name="c", subcore_axis_name="s")

    @functools.partial(
        pl.kernel,
        out_type=jax.ShapeDtypeStruct((batch * seq_len, latent_dim), jnp.float32),
        mesh=mesh,
        scratch_types=[
            pltpu.VMEM((batch, ch), jnp.int32),
            pltpu.VMEM((ch, latent_dim), jnp.float32),
            pltpu.VMEM((2, ch // 2, latent_dim), jnp.float32),
            [pltpu.SemaphoreType.DMA] * 2,
            [pltpu.SemaphoreType.DMA] * 2,
        ],
    )
    def body(tok_hbm, emb_hbm, pos_hbm, out_hbm, idx_v, pbuf, bufs, gsem, ssem):
        wid = lax.axis_index("s") * NC + lax.axis_index("c")
        s0 = wid * ch
        sub = ch // 2
        nt = batch * 2
        pltpu.sync_copy(pos_hbm.at[pl.ds(s0, ch)], pbuf)
        for b in range(batch):
            pltpu.sync_copy(tok_hbm.at[b * NW + wid], idx_v.at[b])

        def gather(t, k):
            b, h = t // 2, t % 2
            return pltpu.async_copy(
                emb_hbm.at[idx_v.at[b, pl.ds(h * sub, sub)]], bufs.at[k], gsem[k]
            )

        gathers = [None] * nt
        stores = [None] * nt
        gathers[0] = gather(0, 0)
        for t in range(nt):
            k = t % 2
            b, h = t // 2, t % 2
            buf = bufs.at[k]
            gathers[t].wait()
            if t + 1 < nt:
                if t >= 1:
                    stores[t - 1].wait()
                gathers[t + 1] = gather(t + 1, 1 - k)

            def row(r, _, buf=buf, h=h):
                for j in range(nsl):
                    sl = pl.ds(j * LANES, LANES)
                    plsc.addupdate(buf.at[r, sl], pbuf[h * sub + r, sl])
                return 0

            lax.fori_loop(0, sub, row, 0)
            stores[t] = pltpu.async_copy(
                buf, out_hbm.at[pl.ds(b * seq_len + s0 + h * sub, sub)], ssem[k]
            )
        stores[nt - 2].wait()
        stores[nt - 1].wait()

    return body


def kernel(tokens, embed_table, pos_table):
    b, s = tokens.shape
    v, d = embed_table.shape
    ch = s // NW
    tok = tokens.reshape(b * NW, ch).astype(jnp.int32)
    out = _make_lookup(b, s, d)(tok, embed_table, pos_table)
    return out.reshape(b, s, d)
